# baseline (speedup 1.0000x reference)
import functools

import jax
import jax.numpy as jnp
from jax import lax
from jax.experimental import pallas as pl
from jax.experimental.pallas import tpu as pltpu
from jax.experimental.pallas import tpu_sc as plsc

NN = 10000
NE = 320000
NPAD = 10112
RPS = NPAD // 16
NTILES = 32
EPT = NE // NTILES

_SC_MESH = plsc.VectorSubcoreMesh(core_axis_name="c", subcore_axis_name="s")
_SC_PARAMS = pltpu.CompilerParams(use_tc_tiling_on_sc=False)


def _deg_body(ei_hbm, ones_hbm, zeros_hbm, out_hbm,
              ones_v, d0, d1, acc_sh, ssem0, ssem1):
    blk = 1000
    nb = EPT // blk
    c = lax.axis_index("c")
    s = lax.axis_index("s")
    wid = c * 16 + s
    e0 = wid * EPT
    pltpu.sync_copy(zeros_hbm.at[pl.ds(s * RPS, RPS)], acc_sh.at[pl.ds(s * RPS, RPS)])
    pltpu.sync_copy(ones_hbm, ones_v)
    plsc.subcore_barrier()

    def body(j2, carry):
        p = e0 + j2 * (2 * blk)

        @pl.when(j2 > 0)
        def _():
            pltpu.make_async_copy(ones_v, acc_sh.at[d0], ssem0).wait()
            pltpu.make_async_copy(ones_v, acc_sh.at[d1], ssem1).wait()

        pltpu.sync_copy(ei_hbm.at[1, pl.ds(p, blk)], d0)
        pltpu.async_copy(ones_v, acc_sh.at[d0], ssem0, add=True)
        pltpu.sync_copy(ei_hbm.at[1, pl.ds(p + blk, blk)], d1)
        pltpu.async_copy(ones_v, acc_sh.at[d1], ssem1, add=True)
        return carry

    lax.fori_loop(0, nb // 2, body, 0)
    pltpu.make_async_copy(ones_v, acc_sh.at[d0], ssem0).wait()
    pltpu.make_async_copy(ones_v, acc_sh.at[d1], ssem1).wait()
    plsc.subcore_barrier()
    pltpu.sync_copy(acc_sh.at[pl.ds(s * RPS, RPS)], out_hbm.at[c, pl.ds(s * RPS, RPS)])


def _make_deg():
    blk = 1000
    return pl.kernel(
        _deg_body,
        out_type=jax.ShapeDtypeStruct((2, NPAD, 8), jnp.float32),
        mesh=_SC_MESH,
        scratch_types=[
            pltpu.VMEM((blk, 8), jnp.float32),
            pltpu.VMEM((blk,), jnp.int32),
            pltpu.VMEM((blk,), jnp.int32),
            pltpu.VMEM_SHARED((NPAD, 8), jnp.float32),
            pltpu.SemaphoreType.DMA,
            pltpu.SemaphoreType.DMA,
        ],
        compiler_params=_SC_PARAMS,
    )


def _agg_body(blk, g_hbm, ei_hbm, zeros_hbm, out_hbm,
              e0b, e1b, r0, r1, acc_sh, is0, is1, gs0, gs1, ss0, ss1):
    nb = EPT // blk
    c = lax.axis_index("c")
    s = lax.axis_index("s")
    wid = c * 16 + s
    e0 = wid * EPT
    pltpu.sync_copy(zeros_hbm.at[pl.ds(s * RPS, RPS)], acc_sh.at[pl.ds(s * RPS, RPS)])
    plsc.subcore_barrier()

    def body(j2, carry):
        p = e0 + j2 * (2 * blk)

        @pl.when(j2 > 0)
        def _():
            pltpu.make_async_copy(r0, acc_sh.at[e0b.at[1]], ss0).wait()
            pltpu.make_async_copy(r1, acc_sh.at[e1b.at[1]], ss1).wait()

        i0 = pltpu.async_copy(ei_hbm.at[:, pl.ds(p, blk)], e0b, is0)
        i1 = pltpu.async_copy(ei_hbm.at[:, pl.ds(p + blk, blk)], e1b, is1)
        i0.wait()
        g0 = pltpu.async_copy(g_hbm.at[e0b.at[0]], r0, gs0)
        i1.wait()
        g1 = pltpu.async_copy(g_hbm.at[e1b.at[0]], r1, gs1)
        g0.wait()
        pltpu.async_copy(r0, acc_sh.at[e0b.at[1]], ss0, add=True)
        g1.wait()
        pltpu.async_copy(r1, acc_sh.at[e1b.at[1]], ss1, add=True)
        return carry

    lax.fori_loop(0, nb // 2, body, 0)
    if nb % 2:
        pltpu.make_async_copy(r0, acc_sh.at[e0b.at[1]], ss0).wait()
        p = e0 + (nb - 1) * blk
        pltpu.sync_copy(ei_hbm.at[:, pl.ds(p, blk)], e0b)
        pltpu.async_copy(g_hbm.at[e0b.at[0]], r0, gs0).wait()
        pltpu.async_copy(r0, acc_sh.at[e0b.at[1]], ss0, add=True)
    pltpu.make_async_copy(r0, acc_sh.at[e0b.at[1]], ss0).wait()
    pltpu.make_async_copy(r1, acc_sh.at[e1b.at[1]], ss1).wait()
    plsc.subcore_barrier()
    pltpu.sync_copy(acc_sh.at[pl.ds(s * RPS, RPS)], out_hbm.at[c, pl.ds(s * RPS, RPS)])


def _make_agg(feat, blk):
    return pl.kernel(
        functools.partial(_agg_body, blk),
        out_type=jax.ShapeDtypeStruct((2, NPAD, feat), jnp.float32),
        mesh=_SC_MESH,
        scratch_types=[
            pltpu.VMEM((2, blk), jnp.int32),
            pltpu.VMEM((2, blk), jnp.int32),
            pltpu.VMEM((blk, feat), jnp.float32),
            pltpu.VMEM((blk, feat), jnp.float32),
            pltpu.VMEM_SHARED((NPAD, feat), jnp.float32),
            pltpu.SemaphoreType.DMA,
            pltpu.SemaphoreType.DMA,
            pltpu.SemaphoreType.DMA,
            pltpu.SemaphoreType.DMA,
            pltpu.SemaphoreType.DMA,
            pltpu.SemaphoreType.DMA,
        ],
        compiler_params=_SC_PARAMS,
    )


def _pre_tc(x_ref, degp_ref, w_ref, ga_ref, gb_ref, dis_ref):
    deg = jnp.sum(degp_ref[...], axis=(0, 2)) * (1.0 / 8.0) + 1.0
    dis = lax.rsqrt(deg)[:, None]
    dis_ref[...] = dis
    h = jnp.dot(x_ref[...] * dis, w_ref[...], preferred_element_type=jnp.float32)
    ga_ref[...] = h[:, :64]
    gb_ref[...] = h[:, 64:]


def _mid1_tc(ta_ref, tb_ref, ga_ref, gb_ref, dis_ref, ba_ref, bb_ref,
             wa_ref, wb_ref, o_ref):
    dis = dis_ref[...]
    za = jnp.maximum((ta_ref[0] + ta_ref[1] + ga_ref[...]) * dis + ba_ref[...], 0.0) * dis
    zb = jnp.maximum((tb_ref[0] + tb_ref[1] + gb_ref[...]) * dis + bb_ref[...], 0.0) * dis
    o_ref[...] = (jnp.dot(za, wa_ref[...], preferred_element_type=jnp.float32)
                  + jnp.dot(zb, wb_ref[...], preferred_element_type=jnp.float32))


def _mid_tc(tp_ref, g_ref, dis_ref, b_ref, w_ref, o_ref):
    dis = dis_ref[...]
    u = (tp_ref[0] + tp_ref[1] + g_ref[...]) * dis + b_ref[...]
    z = jnp.maximum(u, 0.0) * dis
    o_ref[...] = jnp.dot(z, w_ref[...], preferred_element_type=jnp.float32)


def _fin_tc(tp_ref, g_ref, dis_ref, b_ref, wl_ref, bl_ref, o_ref):
    dis = dis_ref[...]
    u = (tp_ref[0] + tp_ref[1] + g_ref[...]) * dis + b_ref[...]
    z = jnp.maximum(u, 0.0)
    o_ref[...] = jnp.dot(z, wl_ref[...], preferred_element_type=jnp.float32) + bl_ref[...]


_RB = 10000


def _pre_call(x, degp, w):
    fi = w.shape[0]
    return pl.pallas_call(
        _pre_tc,
        grid=(NN // _RB,),
        in_specs=[
            pl.BlockSpec((_RB, fi), lambda i: (i, 0)),
            pl.BlockSpec((2, _RB, 8), lambda i: (0, i, 0)),
            pl.BlockSpec((fi, 128), lambda i: (0, 0)),
        ],
        out_specs=[
            pl.BlockSpec((_RB, 64), lambda i: (i, 0)),
            pl.BlockSpec((_RB, 64), lambda i: (i, 0)),
            pl.BlockSpec((_RB, 1), lambda i: (i, 0)),
        ],
        out_shape=[
            jax.ShapeDtypeStruct((NN, 64), jnp.float32),
            jax.ShapeDtypeStruct((NN, 64), jnp.float32),
            jax.ShapeDtypeStruct((NN, 1), jnp.float32),
        ],
    )(x, degp, w)


def _mid1_call(ta, tb, ga, gb, dis_col, b, w):
    fo = w.shape[1]
    return pl.pallas_call(
        _mid1_tc,
        grid=(NN // _RB,),
        in_specs=[
            pl.BlockSpec((2, _RB, 64), lambda i: (0, i, 0)),
            pl.BlockSpec((2, _RB, 64), lambda i: (0, i, 0)),
            pl.BlockSpec((_RB, 64), lambda i: (i, 0)),
            pl.BlockSpec((_RB, 64), lambda i: (i, 0)),
            pl.BlockSpec((_RB, 1), lambda i: (i, 0)),
            pl.BlockSpec((1, 64), lambda i: (0, 0)),
            pl.BlockSpec((1, 64), lambda i: (0, 0)),
            pl.BlockSpec((64, fo), lambda i: (0, 0)),
            pl.BlockSpec((64, fo), lambda i: (0, 0)),
        ],
        out_specs=pl.BlockSpec((_RB, fo), lambda i: (i, 0)),
        out_shape=jax.ShapeDtypeStruct((NN, fo), jnp.float32),
    )(ta, tb, ga, gb, dis_col, b[:, :64], b[:, 64:], w[:64], w[64:])


def _mid_call(tp, g, dis_col, b, w):
    fi, fo = w.shape
    return pl.pallas_call(
        _mid_tc,
        grid=(NN // _RB,),
        in_specs=[
            pl.BlockSpec((2, _RB, fi), lambda i: (0, i, 0)),
            pl.BlockSpec((_RB, fi), lambda i: (i, 0)),
            pl.BlockSpec((_RB, 1), lambda i: (i, 0)),
            pl.BlockSpec((1, fi), lambda i: (0, 0)),
            pl.BlockSpec((fi, fo), lambda i: (0, 0)),
        ],
        out_specs=pl.BlockSpec((_RB, fo), lambda i: (i, 0)),
        out_shape=jax.ShapeDtypeStruct((NN, fo), jnp.float32),
    )(tp, g, dis_col, b, w)


def _fin_call(tp, g, dis_col, b, wl, bl):
    fi, fo = wl.shape
    return pl.pallas_call(
        _fin_tc,
        grid=(NN // _RB,),
        in_specs=[
            pl.BlockSpec((2, _RB, fi), lambda i: (0, i, 0)),
            pl.BlockSpec((_RB, fi), lambda i: (i, 0)),
            pl.BlockSpec((_RB, 1), lambda i: (i, 0)),
            pl.BlockSpec((1, fi), lambda i: (0, 0)),
            pl.BlockSpec((fi, fo), lambda i: (0, 0)),
            pl.BlockSpec((1, fo), lambda i: (0, 0)),
        ],
        out_specs=pl.BlockSpec((_RB, fo), lambda i: (i, 0)),
        out_shape=jax.ShapeDtypeStruct((NN, fo), jnp.float32),
    )(tp, g, dis_col, b, wl, bl)


def kernel(x, edge_index, W1, b1, W2, b2, W3, b3, Wl, bl):
    ei = edge_index.astype(jnp.int32)
    ones8 = jnp.ones((1000, 8), jnp.float32)
    z8 = jnp.zeros((NPAD, 8), jnp.float32)
    z64 = jnp.zeros((NPAD, 64), jnp.float32)
    z32 = jnp.zeros((NPAD, 32), jnp.float32)

    degp = _make_deg()(ei, ones8, z8)
    g1a, g1b, dis_col = _pre_call(x, degp, W1)

    agg64 = _make_agg(64, 400)
    t1a = agg64(g1a, ei, z64)
    t1b = agg64(g1b, ei, z64)
    g2 = _mid1_call(t1a, t1b, g1a, g1b, dis_col, b1.reshape(1, -1), W2)
    t2 = agg64(g2, ei, z64)
    g3 = _mid_call(t2, g2, dis_col, b2.reshape(1, -1), W3)
    t3 = _make_agg(32, 1000)(g3, ei, z32)
    return _fin_call(t3, g3, dis_col, b3.reshape(1, -1), Wl, bl.reshape(1, -1))


# speedup vs baseline: 1.0042x; 1.0042x over previous
import functools

import jax
import jax.numpy as jnp
from jax import lax
from jax.experimental import pallas as pl
from jax.experimental.pallas import tpu as pltpu
from jax.experimental.pallas import tpu_sc as plsc

NN = 10000
NE = 320000
NPAD = 10112
RPS = NPAD // 16
NTILES = 32
EPT = NE // NTILES

_SC_MESH = plsc.VectorSubcoreMesh(core_axis_name="c", subcore_axis_name="s")
_SC_PARAMS = pltpu.CompilerParams(use_tc_tiling_on_sc=False)


def _deg_body(ei_hbm, ones_hbm, zeros_hbm, out_hbm,
              ones_v, d0, d1, acc_sh, ssem0, ssem1):
    blk = 1000
    nb = EPT // blk
    c = lax.axis_index("c")
    s = lax.axis_index("s")
    wid = c * 16 + s
    e0 = wid * EPT
    pltpu.sync_copy(zeros_hbm.at[pl.ds(s * RPS, RPS)], acc_sh.at[pl.ds(s * RPS, RPS)])
    pltpu.sync_copy(ones_hbm, ones_v)
    plsc.subcore_barrier()

    def body(j2, carry):
        p = e0 + j2 * (2 * blk)

        @pl.when(j2 > 0)
        def _():
            pltpu.make_async_copy(ones_v, acc_sh.at[d0], ssem0).wait()
            pltpu.make_async_copy(ones_v, acc_sh.at[d1], ssem1).wait()

        pltpu.sync_copy(ei_hbm.at[1, pl.ds(p, blk)], d0)
        pltpu.async_copy(ones_v, acc_sh.at[d0], ssem0, add=True)
        pltpu.sync_copy(ei_hbm.at[1, pl.ds(p + blk, blk)], d1)
        pltpu.async_copy(ones_v, acc_sh.at[d1], ssem1, add=True)
        return carry

    lax.fori_loop(0, nb // 2, body, 0)
    pltpu.make_async_copy(ones_v, acc_sh.at[d0], ssem0).wait()
    pltpu.make_async_copy(ones_v, acc_sh.at[d1], ssem1).wait()
    plsc.subcore_barrier()
    pltpu.sync_copy(acc_sh.at[pl.ds(s * RPS, RPS)], out_hbm.at[c, pl.ds(s * RPS, RPS)])


def _make_deg():
    blk = 1000
    return pl.kernel(
        _deg_body,
        out_type=jax.ShapeDtypeStruct((2, NPAD, 8), jnp.float32),
        mesh=_SC_MESH,
        scratch_types=[
            pltpu.VMEM((blk, 8), jnp.float32),
            pltpu.VMEM((blk,), jnp.int32),
            pltpu.VMEM((blk,), jnp.int32),
            pltpu.VMEM_SHARED((NPAD, 8), jnp.float32),
            pltpu.SemaphoreType.DMA,
            pltpu.SemaphoreType.DMA,
        ],
        compiler_params=_SC_PARAMS,
    )


def _agg_body(blk, g_hbm, ei_hbm, zeros_hbm, out_hbm,
              e0b, e1b, r0, r1, acc_sh, is0, is1, gs0, gs1, ss0, ss1):
    nb = EPT // blk
    c = lax.axis_index("c")
    s = lax.axis_index("s")
    wid = c * 16 + s
    e0 = wid * EPT
    pltpu.sync_copy(zeros_hbm.at[pl.ds(s * RPS, RPS)], acc_sh.at[pl.ds(s * RPS, RPS)])
    plsc.subcore_barrier()

    def body(j2, carry):
        p = e0 + j2 * (2 * blk)

        @pl.when(j2 > 0)
        def _():
            pltpu.make_async_copy(r0, acc_sh.at[e0b.at[1]], ss0).wait()
            pltpu.make_async_copy(r1, acc_sh.at[e1b.at[1]], ss1).wait()

        i0a = pltpu.async_copy(ei_hbm.at[0, pl.ds(p, blk)], e0b.at[0], is0)
        i0b = pltpu.async_copy(ei_hbm.at[1, pl.ds(p, blk)], e0b.at[1], is0)
        i1a = pltpu.async_copy(ei_hbm.at[0, pl.ds(p + blk, blk)], e1b.at[0], is1)
        i1b = pltpu.async_copy(ei_hbm.at[1, pl.ds(p + blk, blk)], e1b.at[1], is1)
        i0a.wait()
        i0b.wait()
        g0 = pltpu.async_copy(g_hbm.at[e0b.at[0]], r0, gs0)
        i1a.wait()
        i1b.wait()
        g1 = pltpu.async_copy(g_hbm.at[e1b.at[0]], r1, gs1)
        g0.wait()
        pltpu.async_copy(r0, acc_sh.at[e0b.at[1]], ss0, add=True)
        g1.wait()
        pltpu.async_copy(r1, acc_sh.at[e1b.at[1]], ss1, add=True)
        return carry

    lax.fori_loop(0, nb // 2, body, 0)
    if nb % 2:
        pltpu.make_async_copy(r0, acc_sh.at[e0b.at[1]], ss0).wait()
        p = e0 + (nb - 1) * blk
        pltpu.sync_copy(ei_hbm.at[:, pl.ds(p, blk)], e0b)
        pltpu.async_copy(g_hbm.at[e0b.at[0]], r0, gs0).wait()
        pltpu.async_copy(r0, acc_sh.at[e0b.at[1]], ss0, add=True)
    pltpu.make_async_copy(r0, acc_sh.at[e0b.at[1]], ss0).wait()
    pltpu.make_async_copy(r1, acc_sh.at[e1b.at[1]], ss1).wait()
    plsc.subcore_barrier()
    pltpu.sync_copy(acc_sh.at[pl.ds(s * RPS, RPS)], out_hbm.at[c, pl.ds(s * RPS, RPS)])


def _make_agg(feat, blk):
    return pl.kernel(
        functools.partial(_agg_body, blk),
        out_type=jax.ShapeDtypeStruct((2, NPAD, feat), jnp.float32),
        mesh=_SC_MESH,
        scratch_types=[
            pltpu.VMEM((2, blk), jnp.int32),
            pltpu.VMEM((2, blk), jnp.int32),
            pltpu.VMEM((blk, feat), jnp.float32),
            pltpu.VMEM((blk, feat), jnp.float32),
            pltpu.VMEM_SHARED((NPAD, feat), jnp.float32),
            pltpu.SemaphoreType.DMA,
            pltpu.SemaphoreType.DMA,
            pltpu.SemaphoreType.DMA,
            pltpu.SemaphoreType.DMA,
            pltpu.SemaphoreType.DMA,
            pltpu.SemaphoreType.DMA,
        ],
        compiler_params=_SC_PARAMS,
    )


def _pre_tc(x_ref, degp_ref, w_ref, ga_ref, gb_ref, dis_ref):
    deg = jnp.sum(degp_ref[...], axis=(0, 2)) * (1.0 / 8.0) + 1.0
    dis = lax.rsqrt(deg)[:, None]
    dis_ref[...] = dis
    h = jnp.dot(x_ref[...] * dis, w_ref[...], preferred_element_type=jnp.float32)
    ga_ref[...] = h[:, :64]
    gb_ref[...] = h[:, 64:]


def _mid1_tc(ta_ref, tb_ref, ga_ref, gb_ref, dis_ref, ba_ref, bb_ref,
             wa_ref, wb_ref, o_ref):
    dis = dis_ref[...]
    za = jnp.maximum((ta_ref[0] + ta_ref[1] + ga_ref[...]) * dis + ba_ref[...], 0.0) * dis
    zb = jnp.maximum((tb_ref[0] + tb_ref[1] + gb_ref[...]) * dis + bb_ref[...], 0.0) * dis
    o_ref[...] = (jnp.dot(za, wa_ref[...], preferred_element_type=jnp.float32)
                  + jnp.dot(zb, wb_ref[...], preferred_element_type=jnp.float32))


def _mid_tc(tp_ref, g_ref, dis_ref, b_ref, w_ref, o_ref):
    dis = dis_ref[...]
    u = (tp_ref[0] + tp_ref[1] + g_ref[...]) * dis + b_ref[...]
    z = jnp.maximum(u, 0.0) * dis
    o_ref[...] = jnp.dot(z, w_ref[...], preferred_element_type=jnp.float32)


def _fin_tc(tp_ref, g_ref, dis_ref, b_ref, wl_ref, bl_ref, o_ref):
    dis = dis_ref[...]
    u = (tp_ref[0] + tp_ref[1] + g_ref[...]) * dis + b_ref[...]
    z = jnp.maximum(u, 0.0)
    o_ref[...] = jnp.dot(z, wl_ref[...], preferred_element_type=jnp.float32) + bl_ref[...]


_RB = 2000


def _pre_call(x, degp, w):
    fi = w.shape[0]
    return pl.pallas_call(
        _pre_tc,
        grid=(NN // _RB,),
        in_specs=[
            pl.BlockSpec((_RB, fi), lambda i: (i, 0)),
            pl.BlockSpec((2, _RB, 8), lambda i: (0, i, 0)),
            pl.BlockSpec((fi, 128), lambda i: (0, 0)),
        ],
        out_specs=[
            pl.BlockSpec((_RB, 64), lambda i: (i, 0)),
            pl.BlockSpec((_RB, 64), lambda i: (i, 0)),
            pl.BlockSpec((_RB, 1), lambda i: (i, 0)),
        ],
        out_shape=[
            jax.ShapeDtypeStruct((NN, 64), jnp.float32),
            jax.ShapeDtypeStruct((NN, 64), jnp.float32),
            jax.ShapeDtypeStruct((NN, 1), jnp.float32),
        ],
    )(x, degp, w)


def _mid1_call(ta, tb, ga, gb, dis_col, b, w):
    fo = w.shape[1]
    return pl.pallas_call(
        _mid1_tc,
        grid=(NN // _RB,),
        in_specs=[
            pl.BlockSpec((2, _RB, 64), lambda i: (0, i, 0)),
            pl.BlockSpec((2, _RB, 64), lambda i: (0, i, 0)),
            pl.BlockSpec((_RB, 64), lambda i: (i, 0)),
            pl.BlockSpec((_RB, 64), lambda i: (i, 0)),
            pl.BlockSpec((_RB, 1), lambda i: (i, 0)),
            pl.BlockSpec((1, 64), lambda i: (0, 0)),
            pl.BlockSpec((1, 64), lambda i: (0, 0)),
            pl.BlockSpec((64, fo), lambda i: (0, 0)),
            pl.BlockSpec((64, fo), lambda i: (0, 0)),
        ],
        out_specs=pl.BlockSpec((_RB, fo), lambda i: (i, 0)),
        out_shape=jax.ShapeDtypeStruct((NN, fo), jnp.float32),
    )(ta, tb, ga, gb, dis_col, b[:, :64], b[:, 64:], w[:64], w[64:])


def _mid_call(tp, g, dis_col, b, w):
    fi, fo = w.shape
    return pl.pallas_call(
        _mid_tc,
        grid=(NN // _RB,),
        in_specs=[
            pl.BlockSpec((2, _RB, fi), lambda i: (0, i, 0)),
            pl.BlockSpec((_RB, fi), lambda i: (i, 0)),
            pl.BlockSpec((_RB, 1), lambda i: (i, 0)),
            pl.BlockSpec((1, fi), lambda i: (0, 0)),
            pl.BlockSpec((fi, fo), lambda i: (0, 0)),
        ],
        out_specs=pl.BlockSpec((_RB, fo), lambda i: (i, 0)),
        out_shape=jax.ShapeDtypeStruct((NN, fo), jnp.float32),
    )(tp, g, dis_col, b, w)


def _fin_call(tp, g, dis_col, b, wl, bl):
    fi, fo = wl.shape
    return pl.pallas_call(
        _fin_tc,
        grid=(NN // _RB,),
        in_specs=[
            pl.BlockSpec((2, _RB, fi), lambda i: (0, i, 0)),
            pl.BlockSpec((_RB, fi), lambda i: (i, 0)),
            pl.BlockSpec((_RB, 1), lambda i: (i, 0)),
            pl.BlockSpec((1, fi), lambda i: (0, 0)),
            pl.BlockSpec((fi, fo), lambda i: (0, 0)),
            pl.BlockSpec((1, fo), lambda i: (0, 0)),
        ],
        out_specs=pl.BlockSpec((_RB, fo), lambda i: (i, 0)),
        out_shape=jax.ShapeDtypeStruct((NN, fo), jnp.float32),
    )(tp, g, dis_col, b, wl, bl)


def kernel(x, edge_index, W1, b1, W2, b2, W3, b3, Wl, bl):
    ei = edge_index.astype(jnp.int32)
    ones8 = jnp.ones((1000, 8), jnp.float32)
    z8 = jnp.zeros((NPAD, 8), jnp.float32)
    z64 = jnp.zeros((NPAD, 64), jnp.float32)
    z32 = jnp.zeros((NPAD, 32), jnp.float32)

    degp = _make_deg()(ei, ones8, z8)
    g1a, g1b, dis_col = _pre_call(x, degp, W1)

    agg64 = _make_agg(64, 400)
    t1a = agg64(g1a, ei, z64)
    t1b = agg64(g1b, ei, z64)
    g2 = _mid1_call(t1a, t1b, g1a, g1b, dis_col, b1.reshape(1, -1), W2)
    t2 = agg64(g2, ei, z64)
    g3 = _mid_call(t2, g2, dis_col, b2.reshape(1, -1), W3)
    t3 = _make_agg(32, 1000)(g3, ei, z32)
    return _fin_call(t3, g3, dis_col, b3.reshape(1, -1), Wl, bl.reshape(1, -1))


# speedup vs baseline: 1.1595x; 1.1546x over previous
import functools

import jax
import jax.numpy as jnp
from jax import lax
from jax.experimental import pallas as pl
from jax.experimental.pallas import tpu as pltpu
from jax.experimental.pallas import tpu_sc as plsc

NN = 10000
NE = 320000
NPAD = 10112
RPS = NPAD // 16
NTILES = 32
EPT = NE // NTILES

_SC_MESH = plsc.VectorSubcoreMesh(core_axis_name="c", subcore_axis_name="s")
_SC_PARAMS = pltpu.CompilerParams(use_tc_tiling_on_sc=False)


def _deg_body(ei_hbm, ones_hbm, zeros_hbm, out_hbm,
              ones_v, d0, d1, acc_sh, ssem0, ssem1):
    blk = 1000
    nb = EPT // blk
    c = lax.axis_index("c")
    s = lax.axis_index("s")
    wid = c * 16 + s
    e0 = wid * EPT
    pltpu.sync_copy(zeros_hbm.at[pl.ds(s * RPS, RPS)], acc_sh.at[pl.ds(s * RPS, RPS)])
    pltpu.sync_copy(ones_hbm, ones_v)
    plsc.subcore_barrier()

    def body(j2, carry):
        p = e0 + j2 * (2 * blk)

        @pl.when(j2 > 0)
        def _():
            pltpu.make_async_copy(ones_v, acc_sh.at[d0], ssem0).wait()
            pltpu.make_async_copy(ones_v, acc_sh.at[d1], ssem1).wait()

        pltpu.sync_copy(ei_hbm.at[1, pl.ds(p, blk)], d0)
        pltpu.async_copy(ones_v, acc_sh.at[d0], ssem0, add=True)
        pltpu.sync_copy(ei_hbm.at[1, pl.ds(p + blk, blk)], d1)
        pltpu.async_copy(ones_v, acc_sh.at[d1], ssem1, add=True)
        return carry

    lax.fori_loop(0, nb // 2, body, 0)
    pltpu.make_async_copy(ones_v, acc_sh.at[d0], ssem0).wait()
    pltpu.make_async_copy(ones_v, acc_sh.at[d1], ssem1).wait()
    plsc.subcore_barrier()
    pltpu.sync_copy(acc_sh.at[pl.ds(s * RPS, RPS)], out_hbm.at[c, pl.ds(s * RPS, RPS)])


def _make_deg():
    blk = 1000
    return pl.kernel(
        _deg_body,
        out_type=jax.ShapeDtypeStruct((2, NPAD, 8), jnp.float32),
        mesh=_SC_MESH,
        scratch_types=[
            pltpu.VMEM((blk, 8), jnp.float32),
            pltpu.VMEM((blk,), jnp.int32),
            pltpu.VMEM((blk,), jnp.int32),
            pltpu.VMEM_SHARED((NPAD, 8), jnp.float32),
            pltpu.SemaphoreType.DMA,
            pltpu.SemaphoreType.DMA,
        ],
        compiler_params=_SC_PARAMS,
    )


def _agg_body(blk, g_hbm, ei_hbm, zeros_hbm, out_hbm,
              e0b, e1b, r0, r1, acc_sh, is0, is1, gs0, gs1, ss0, ss1):
    nb = EPT // blk
    c = lax.axis_index("c")
    s = lax.axis_index("s")
    wid = c * 16 + s
    e0 = wid * EPT
    pltpu.sync_copy(zeros_hbm.at[pl.ds(s * RPS, RPS)], acc_sh.at[pl.ds(s * RPS, RPS)])
    plsc.subcore_barrier()

    def body(j2, carry):
        p = e0 + j2 * (2 * blk)

        @pl.when(j2 > 0)
        def _():
            pltpu.make_async_copy(r0, acc_sh.at[e0b.at[1]], ss0).wait()
            pltpu.make_async_copy(r1, acc_sh.at[e1b.at[1]], ss1).wait()

        i0a = pltpu.async_copy(ei_hbm.at[0, pl.ds(p, blk)], e0b.at[0], is0)
        i0b = pltpu.async_copy(ei_hbm.at[1, pl.ds(p, blk)], e0b.at[1], is0)
        i1a = pltpu.async_copy(ei_hbm.at[0, pl.ds(p + blk, blk)], e1b.at[0], is1)
        i1b = pltpu.async_copy(ei_hbm.at[1, pl.ds(p + blk, blk)], e1b.at[1], is1)
        i0a.wait()
        i0b.wait()
        g0 = pltpu.async_copy(g_hbm.at[e0b.at[0]], r0, gs0)
        i1a.wait()
        i1b.wait()
        g1 = pltpu.async_copy(g_hbm.at[e1b.at[0]], r1, gs1)
        g0.wait()
        pltpu.async_copy(r0, acc_sh.at[e0b.at[1]], ss0, add=True)
        g1.wait()
        pltpu.async_copy(r1, acc_sh.at[e1b.at[1]], ss1, add=True)
        return carry

    lax.fori_loop(0, nb // 2, body, 0)
    if nb % 2:
        pltpu.make_async_copy(r0, acc_sh.at[e0b.at[1]], ss0).wait()
        p = e0 + (nb - 1) * blk
        pltpu.sync_copy(ei_hbm.at[:, pl.ds(p, blk)], e0b)
        pltpu.async_copy(g_hbm.at[e0b.at[0]], r0, gs0).wait()
        pltpu.async_copy(r0, acc_sh.at[e0b.at[1]], ss0, add=True)
    pltpu.make_async_copy(r0, acc_sh.at[e0b.at[1]], ss0).wait()
    pltpu.make_async_copy(r1, acc_sh.at[e1b.at[1]], ss1).wait()
    plsc.subcore_barrier()
    pltpu.sync_copy(acc_sh.at[pl.ds(s * RPS, RPS)], out_hbm.at[c, pl.ds(s * RPS, RPS)])


def _make_agg(feat, blk):
    return pl.kernel(
        functools.partial(_agg_body, blk),
        out_type=jax.ShapeDtypeStruct((2, NPAD, feat), jnp.float32),
        mesh=_SC_MESH,
        scratch_types=[
            pltpu.VMEM((2, blk), jnp.int32),
            pltpu.VMEM((2, blk), jnp.int32),
            pltpu.VMEM((blk, feat), jnp.float32),
            pltpu.VMEM((blk, feat), jnp.float32),
            pltpu.VMEM_SHARED((NPAD, feat), jnp.float32),
            pltpu.SemaphoreType.DMA,
            pltpu.SemaphoreType.DMA,
            pltpu.SemaphoreType.DMA,
            pltpu.SemaphoreType.DMA,
            pltpu.SemaphoreType.DMA,
            pltpu.SemaphoreType.DMA,
        ],
        compiler_params=_SC_PARAMS,
    )


def _pre_tc(x_ref, degp_ref, w_ref, ga_ref, gb_ref, dis_ref):
    deg = jnp.sum(degp_ref[...], axis=(0, 2)) * (1.0 / 8.0) + 1.0
    dis = (1.0 / jnp.sqrt(deg))[:, None]
    dis_ref[...] = dis
    h = jnp.dot(x_ref[...], w_ref[...], preferred_element_type=jnp.float32) * dis
    ga_ref[...] = h[:, :64]
    gb_ref[...] = h[:, 64:]


def _mid1_tc(ta_ref, tb_ref, ga_ref, gb_ref, dis_ref, ba_ref, bb_ref,
             wa_ref, wb_ref, o_ref):
    dis = dis_ref[...]
    za = jnp.maximum((ta_ref[0] + ta_ref[1] + ga_ref[...]) * dis + ba_ref[...], 0.0)
    zb = jnp.maximum((tb_ref[0] + tb_ref[1] + gb_ref[...]) * dis + bb_ref[...], 0.0)
    o_ref[...] = (jnp.dot(za, wa_ref[...], preferred_element_type=jnp.float32)
                  + jnp.dot(zb, wb_ref[...], preferred_element_type=jnp.float32)) * dis


def _mid_tc(tp_ref, g_ref, dis_ref, b_ref, w_ref, o_ref):
    dis = dis_ref[...]
    u = (tp_ref[0] + tp_ref[1] + g_ref[...]) * dis + b_ref[...]
    z = jnp.maximum(u, 0.0)
    o_ref[...] = jnp.dot(z, w_ref[...], preferred_element_type=jnp.float32) * dis


def _fin_tc(tp_ref, g_ref, dis_ref, b_ref, wl_ref, bl_ref, o_ref):
    dis = dis_ref[...]
    u = (tp_ref[0] + tp_ref[1] + g_ref[...]) * dis + b_ref[...]
    z = jnp.maximum(u, 0.0)
    o_ref[...] = jnp.dot(z, wl_ref[...], preferred_element_type=jnp.float32) + bl_ref[...]


_RB = 2000


def _pre_call(x, degp, w):
    fi = w.shape[0]
    return pl.pallas_call(
        _pre_tc,
        grid=(NN // _RB,),
        in_specs=[
            pl.BlockSpec((_RB, fi), lambda i: (i, 0)),
            pl.BlockSpec((2, _RB, 8), lambda i: (0, i, 0)),
            pl.BlockSpec((fi, 128), lambda i: (0, 0)),
        ],
        out_specs=[
            pl.BlockSpec((_RB, 64), lambda i: (i, 0)),
            pl.BlockSpec((_RB, 64), lambda i: (i, 0)),
            pl.BlockSpec((_RB, 1), lambda i: (i, 0)),
        ],
        out_shape=[
            jax.ShapeDtypeStruct((NN, 64), jnp.float32),
            jax.ShapeDtypeStruct((NN, 64), jnp.float32),
            jax.ShapeDtypeStruct((NN, 1), jnp.float32),
        ],
    )(x, degp, w)


def _mid1_call(ta, tb, ga, gb, dis_col, b, w):
    fo = w.shape[1]
    return pl.pallas_call(
        _mid1_tc,
        grid=(NN // _RB,),
        in_specs=[
            pl.BlockSpec((2, _RB, 64), lambda i: (0, i, 0)),
            pl.BlockSpec((2, _RB, 64), lambda i: (0, i, 0)),
            pl.BlockSpec((_RB, 64), lambda i: (i, 0)),
            pl.BlockSpec((_RB, 64), lambda i: (i, 0)),
            pl.BlockSpec((_RB, 1), lambda i: (i, 0)),
            pl.BlockSpec((1, 64), lambda i: (0, 0)),
            pl.BlockSpec((1, 64), lambda i: (0, 0)),
            pl.BlockSpec((64, fo), lambda i: (0, 0)),
            pl.BlockSpec((64, fo), lambda i: (0, 0)),
        ],
        out_specs=pl.BlockSpec((_RB, fo), lambda i: (i, 0)),
        out_shape=jax.ShapeDtypeStruct((NN, fo), jnp.float32),
    )(ta, tb, ga, gb, dis_col, b[:, :64], b[:, 64:], w[:64], w[64:])


def _mid_call(tp, g, dis_col, b, w):
    fi, fo = w.shape
    return pl.pallas_call(
        _mid_tc,
        grid=(NN // _RB,),
        in_specs=[
            pl.BlockSpec((2, _RB, fi), lambda i: (0, i, 0)),
            pl.BlockSpec((_RB, fi), lambda i: (i, 0)),
            pl.BlockSpec((_RB, 1), lambda i: (i, 0)),
            pl.BlockSpec((1, fi), lambda i: (0, 0)),
            pl.BlockSpec((fi, fo), lambda i: (0, 0)),
        ],
        out_specs=pl.BlockSpec((_RB, fo), lambda i: (i, 0)),
        out_shape=jax.ShapeDtypeStruct((NN, fo), jnp.float32),
    )(tp, g, dis_col, b, w)


def _fin_call(tp, g, dis_col, b, wl, bl):
    fi, fo = wl.shape
    return pl.pallas_call(
        _fin_tc,
        grid=(NN // _RB,),
        in_specs=[
            pl.BlockSpec((2, _RB, fi), lambda i: (0, i, 0)),
            pl.BlockSpec((_RB, fi), lambda i: (i, 0)),
            pl.BlockSpec((_RB, 1), lambda i: (i, 0)),
            pl.BlockSpec((1, fi), lambda i: (0, 0)),
            pl.BlockSpec((fi, fo), lambda i: (0, 0)),
            pl.BlockSpec((1, fo), lambda i: (0, 0)),
        ],
        out_specs=pl.BlockSpec((_RB, fo), lambda i: (i, 0)),
        out_shape=jax.ShapeDtypeStruct((NN, fo), jnp.float32),
    )(tp, g, dis_col, b, wl, bl)


def kernel(x, edge_index, W1, b1, W2, b2, W3, b3, Wl, bl):
    ei = edge_index.astype(jnp.int32)
    ones8 = jnp.ones((1000, 8), jnp.float32)
    z8 = jnp.zeros((NPAD, 8), jnp.float32)
    z64 = jnp.zeros((NPAD, 64), jnp.float32)
    z32 = jnp.zeros((NPAD, 32), jnp.float32)

    degp = _make_deg()(ei, ones8, z8)
    g1a, g1b, dis_col = _pre_call(x, degp, W1)

    agg64 = _make_agg(64, 400)
    t1a = agg64(g1a, ei, z64)
    t1b = agg64(g1b, ei, z64)
    g2 = _mid1_call(t1a, t1b, g1a, g1b, dis_col, b1.reshape(1, -1), W2)
    t2 = agg64(g2, ei, z64)
    g3 = _mid_call(t2, g2, dis_col, b2.reshape(1, -1), W3)
    t3 = _make_agg(32, 1000)(g3, ei, z32)
    return _fin_call(t3, g3, dis_col, b3.reshape(1, -1), Wl, bl.reshape(1, -1))
